# baseline (speedup 1.0000x reference)
import functools

import jax
import jax.numpy as jnp
from jax import lax
from jax.experimental import pallas as pl
from jax.experimental.pallas import tpu as pltpu
from jax.experimental.pallas import tpu_sc as plsc

B, T, V = 1024, 50, 1000
N = B * T
NC, NS, L = 2, 16, 16
NW = NC * NS
PER = N // NW
CHUNK = 80
NCHUNK = PER // CHUNK
BLK = 128
CPW = B // NW

_mesh = plsc.VectorSubcoreMesh(core_axis_name="c", subcore_axis_name="s")


@functools.partial(
    pl.kernel,
    out_type=[
        jax.ShapeDtypeStruct((NW, L), jnp.float32),
        jax.ShapeDtypeStruct((NW, L), jnp.float32),
    ],
    mesh=_mesh,
    scratch_types=[
        pltpu.VMEM((T, BLK), jnp.int32),
        pltpu.VMEM((T, BLK), jnp.float32),
        pltpu.VMEM((PER,), jnp.int32),
        pltpu.VMEM((PER,), jnp.float32),
        pltpu.VMEM((L,), jnp.float32),
        pltpu.VMEM((L,), jnp.float32),
        pltpu.SemaphoreType.DMA,
    ],
)
def _sc_gather(flat_hbm, tgt_hbm, mask_hbm, prod_out, mask_out,
               tgt_v, mask_v, idx_v, vals_v, acc_v, macc_v, sem):
    wid = lax.axis_index("s") * NC + lax.axis_index("c")
    blk = wid // 4
    col0 = pl.multiple_of((wid % 4) * CPW, 8)

    bcol = pl.multiple_of(blk * BLK, BLK)
    pltpu.sync_copy(tgt_hbm.at[:, pl.ds(bcol, BLK)], tgt_v)
    pltpu.sync_copy(mask_hbm.at[:, pl.ds(bcol, BLK)], mask_v)

    lane = lax.iota(jnp.int32, L)

    @pl.loop(0, T)
    def _(j):
        for s in range(2):
            c = pl.multiple_of(col0 + s * L, 8)
            t16 = tgt_v[j, pl.ds(c, L)]
            idx16 = (j * (V * B) + (t16 >> 3) * (8 * BLK * 8)
                     + blk * (8 * BLK) + (t16 & 7) * BLK + c + lane)
            idx_v[pl.ds(pl.multiple_of(j * 2 * L, 8) + s * L, L)] = idx16

    def _chunk_copy(c):
        sl = pl.ds(pl.multiple_of(c * CHUNK, 8), CHUNK)
        return pltpu.make_async_copy(
            flat_hbm.at[idx_v.at[sl]], vals_v.at[sl], sem)

    @pl.loop(0, NCHUNK)
    def _(c):
        _chunk_copy(c).start()

    @pl.loop(0, NCHUNK)
    def _(c):
        _chunk_copy(c).wait()

    acc_v[...] = jnp.zeros((L,), jnp.float32)
    macc_v[...] = jnp.zeros((L,), jnp.float32)

    @pl.loop(0, T)
    def _(j):
        for s in range(2):
            c = pl.multiple_of(col0 + s * L, 8)
            m = mask_v[j, pl.ds(c, L)]
            v = vals_v[pl.ds(pl.multiple_of(j * 2 * L, 8) + s * L, L)]
            acc_v[...] += v * m
            macc_v[...] += m

    pltpu.sync_copy(acc_v, prod_out.at[wid])
    pltpu.sync_copy(macc_v, mask_out.at[wid])


def _finish_body(p_ref, m_ref, o_ref):
    s = -jnp.sum(p_ref[...]) / jnp.sum(m_ref[...])
    o_ref[...] = jnp.full((1, 1), s, jnp.float32)


_finish = pl.pallas_call(
    _finish_body,
    out_shape=jax.ShapeDtypeStruct((1, 1), jnp.float32),
)


def kernel(input, target, mask):
    x1 = (input.transpose(1, 2, 0)
          .reshape(T, V // 8, 8, B // 128, 128)
          .transpose(0, 1, 3, 2, 4)
          .reshape(-1))
    prod_p, mask_p = _sc_gather(x1, target.T, mask.T)
    return _finish(prod_p, mask_p)[0, 0]
